# Initial kernel scaffold; baseline (speedup 1.0000x reference)
import jax
import jax.numpy as jnp
from jax.experimental import pallas as pl


def kernel(input_point_cloud, model_keypoints, cad_models, W1, W2, W3, W4, b4):
    raise NotImplementedError("write your pallas kernel here")


# speedup vs baseline: 15.6072x; 15.6072x over previous
import functools

import jax
import jax.numpy as jnp
from jax.experimental import pallas as pl
from jax.experimental.pallas import tpu as pltpu

B, M, NKP, NCAD, K_NN = 16, 1024, 16, 1024, 16
_F32 = jnp.float32
_HI = jax.lax.Precision.HIGHEST
_BIG = 3.0e38


def _dot(a, b, precision=_HI):
    return jax.lax.dot_general(a, b, (((1,), (0,)), ((), ())),
                               precision=precision,
                               preferred_element_type=_F32)


def _detect_body(x_ref, xt_ref, w1_ref, w2_ref, w3_ref, w4_ref, b4_ref,
                 out_ref):
    pos = x_ref[...]
    posT = xt_ref[...]
    r0, r1, r2 = pos[0:1, :], pos[1:2, :], pos[2:3, :]
    sq_row = (r0 * r0 + r1 * r1) + r2 * r2
    sq_col = jnp.sum(posT * posT, axis=1, keepdims=True)
    dot = _dot(posT, pos, precision=jax.lax.Precision.DEFAULT)
    d2 = (sq_col + sq_row) - 2.0 * dot

    idx = jax.lax.broadcasted_iota(jnp.int32, (M, M), 1)
    work = d2
    acc = jnp.zeros((M, M), dtype=_F32)
    for _ in range(K_NN):
        mval = jnp.min(work, axis=1, keepdims=True)
        cand = jnp.where(work <= mval, idx, jnp.int32(M))
        jstar = jnp.min(cand, axis=1, keepdims=True)
        sel = idx == jstar
        acc = acc + sel.astype(_F32)
        work = jnp.where(sel, _BIG, work)

    nbr = _dot(acc, posT)
    feats = jnp.concatenate([posT, K_NN * posT - nbr], axis=1)

    lo = jax.lax.Precision.DEFAULT
    h = jnp.maximum(_dot(feats, w1_ref[...], precision=lo), 0.0)
    h = jnp.maximum(_dot(h, w2_ref[...], precision=lo), 0.0)
    h = jnp.maximum(_dot(h, w3_ref[...], precision=lo), 0.0)
    g = jnp.max(h, axis=0, keepdims=True)
    out_ref[...] = _dot(g, w4_ref[...], precision=lo) + b4_ref[...]


def _recip(x):
    r = 1.0 / x
    r = r * (2.0 - x * r)
    r = r * (2.0 - x * r)
    return r


def _rsqrt_acc(x):
    r = jax.lax.rsqrt(x)
    r = r * (1.5 - 0.5 * x * r * r)
    r = r * (1.5 - 0.5 * x * r * r)
    return r


def _bf(x):
    return x.astype(jnp.bfloat16).astype(_F32)


def _jacobi_eig4(nmat):
    a = list(nmat)
    v = [jnp.full_like(a[0], 1.0 if i % 5 == 0 else 0.0) for i in range(16)]
    for _ in range(6):
        for (p, q) in ((0, 1), (0, 2), (0, 3), (1, 2), (1, 3), (2, 3)):
            app, aqq, apq = a[4 * p + p], a[4 * q + q], a[4 * p + q]
            theta = (aqq - app) * _recip(2.0 * apq)
            th2p1 = theta * theta + 1.0
            t_raw = jnp.sign(theta) * _recip(
                jnp.abs(theta) + th2p1 * _rsqrt_acc(th2p1))
            t = jnp.where((jnp.abs(apq) > 0.0) & (th2p1 <= 3.0e38),
                          t_raw, 0.0)
            c = _rsqrt_acc(t * t + 1.0)
            s = t * c
            new_pp = app - t * apq
            new_qq = aqq + t * apq
            for k in range(4):
                if k in (p, q):
                    continue
                akp, akq = a[4 * k + p], a[4 * k + q]
                np_ = c * akp - s * akq
                nq_ = s * akp + c * akq
                a[4 * k + p] = np_
                a[4 * p + k] = np_
                a[4 * k + q] = nq_
                a[4 * q + k] = nq_
            a[4 * p + p] = new_pp
            a[4 * q + q] = new_qq
            zero = jnp.zeros_like(apq)
            a[4 * p + q] = zero
            a[4 * q + p] = zero
            for k in range(4):
                vkp, vkq = v[4 * k + p], v[4 * k + q]
                v[4 * k + p] = c * vkp - s * vkq
                v[4 * k + q] = s * vkp + c * vkq
    diag = [a[0], a[5], a[10], a[15]]
    return diag, v


def _kabsch_body(det_ref, mk_ref, cad_ref, pred_ref, r_ref, t_ref):
    det = det_ref[...]
    src = mk_ref[...]

    scol = []
    sc = []
    for i in range(3):
        row = src[i:i + 1, :]
        m = jnp.mean(row, axis=1, keepdims=True)
        sc.append(m)
        scol.append(row - m)

    tcen = []
    tc = []
    for j in range(3):
        blk = det[:, NKP * j:NKP * (j + 1)]
        m = jnp.mean(blk, axis=1, keepdims=True)
        tc.append(m)
        tcen.append(blk - m)

    sb = [_bf(scol[i]) for i in range(3)]
    tb = [_bf(tcen[j]) for j in range(3)]
    h = [[jnp.sum(sb[i] * tb[j], axis=1, keepdims=True)
          for j in range(3)] for i in range(3)]

    sxx, sxy, sxz = h[0][0], h[0][1], h[0][2]
    syx, syy, syz = h[1][0], h[1][1], h[1][2]
    szx, szy, szz = h[2][0], h[2][1], h[2][2]
    nm = [None] * 16
    nm[0] = sxx + syy + szz
    nm[1] = syz - szy
    nm[2] = szx - sxz
    nm[3] = sxy - syx
    nm[5] = sxx - syy - szz
    nm[6] = sxy + syx
    nm[7] = szx + sxz
    nm[10] = -sxx + syy - szz
    nm[11] = syz + szy
    nm[15] = -sxx - syy + szz
    nm[4], nm[8], nm[12] = nm[1], nm[2], nm[3]
    nm[9], nm[13], nm[14] = nm[6], nm[7], nm[11]

    diag, v = _jacobi_eig4(nm)
    mx = jnp.maximum(jnp.maximum(diag[0], diag[1]),
                     jnp.maximum(diag[2], diag[3]))
    is0 = diag[0] >= mx
    is1 = diag[1] >= mx
    is2 = diag[2] >= mx
    quat = []
    for i in range(4):
        col = jnp.where(is0, v[4 * i + 0],
                        jnp.where(is1, v[4 * i + 1],
                                  jnp.where(is2, v[4 * i + 2], v[4 * i + 3])))
        quat.append(col)
    qw, qx, qy, qz = quat
    nrm = qw * qw + qx * qx + qy * qy + qz * qz
    s2 = 2.0 * _recip(nrm)
    r00 = 1.0 - s2 * (qy * qy + qz * qz)
    r01 = s2 * (qx * qy - qw * qz)
    r02 = s2 * (qx * qz + qw * qy)
    r10 = s2 * (qx * qy + qw * qz)
    r11 = 1.0 - s2 * (qx * qx + qz * qz)
    r12 = s2 * (qy * qz - qw * qx)
    r20 = s2 * (qx * qz - qw * qy)
    r21 = s2 * (qy * qz + qw * qx)
    r22 = 1.0 - s2 * (qx * qx + qy * qy)
    rmat = [[r00, r01, r02], [r10, r11, r12], [r20, r21, r22]]

    rb = [[_bf(rmat[i][j]) for j in range(3)] for i in range(3)]
    scb = [_bf(sc[j]) for j in range(3)]
    tvec = [tc[i] - ((rb[i][0] * scb[0] + rb[i][1] * scb[1])
                     + rb[i][2] * scb[2]) for i in range(3)]

    cad = cad_ref[...]
    cb = [_bf(cad[j:j + 1, :]) for j in range(3)]
    for i in range(3):
        row = ((rb[i][0] * cb[0] + rb[i][1] * cb[1])
               + rb[i][2] * cb[2]) + tvec[i]
        pred_ref[:, i, :] = row
    r_ref[...] = jnp.concatenate(
        [rmat[i][j] for i in range(3) for j in range(3)], axis=1)
    t_ref[...] = jnp.concatenate(tvec, axis=1)


@functools.partial(jax.jit, static_argnames=())
def kernel(input_point_cloud, model_keypoints, cad_models, W1, W2, W3, W4,
           b4):
    x = input_point_cloud.astype(_F32)
    xt = jnp.swapaxes(x, 1, 2)
    b4r = b4.reshape(1, 3 * NKP).astype(_F32)

    wspec = lambda shp: pl.BlockSpec(shp, lambda b: (0, 0))
    detected = pl.pallas_call(
        _detect_body,
        grid=(B,),
        in_specs=[
            pl.BlockSpec((None, 3, M), lambda b: (b, 0, 0)),
            pl.BlockSpec((None, M, 3), lambda b: (b, 0, 0)),
            wspec((6, 32)), wspec((32, 64)), wspec((64, 128)),
            wspec((128, 3 * NKP)), wspec((1, 3 * NKP)),
        ],
        out_specs=pl.BlockSpec((None, 1, 3 * NKP), lambda b: (b, 0, 0)),
        out_shape=jax.ShapeDtypeStruct((B, 1, 3 * NKP), _F32),
        compiler_params=pltpu.CompilerParams(
            dimension_semantics=("arbitrary",)),
    )(x, xt, W1.astype(_F32), W2.astype(_F32), W3.astype(_F32),
      W4.astype(_F32), b4r)
    detected = detected.reshape(B, 3 * NKP)

    mk = model_keypoints[0].astype(_F32)
    cad = cad_models[0].astype(_F32)
    pred, r9, t3 = pl.pallas_call(
        _kabsch_body,
        in_specs=[pl.BlockSpec((B, 3 * NKP), lambda: (0, 0)),
                  pl.BlockSpec((3, NKP), lambda: (0, 0)),
                  pl.BlockSpec((3, NCAD), lambda: (0, 0))],
        out_specs=[pl.BlockSpec((B, 3, NCAD), lambda: (0, 0, 0)),
                   pl.BlockSpec((B, 9), lambda: (0, 0)),
                   pl.BlockSpec((B, 3), lambda: (0, 0))],
        out_shape=[jax.ShapeDtypeStruct((B, 3, NCAD), _F32),
                   jax.ShapeDtypeStruct((B, 9), _F32),
                   jax.ShapeDtypeStruct((B, 3), _F32)],
    )(detected, mk, cad)

    return (pred, detected.reshape(B, 3, NKP), r9.reshape(B, 3, 3),
            t3.reshape(B, 3, 1))


# speedup vs baseline: 15.6087x; 1.0001x over previous
import functools

import jax
import jax.numpy as jnp
from jax.experimental import pallas as pl
from jax.experimental.pallas import tpu as pltpu

B, M, NKP, NCAD, K_NN = 16, 1024, 16, 1024, 16
_F32 = jnp.float32
_HI = jax.lax.Precision.HIGHEST
_BIG = 3.0e38


def _dot(a, b, precision=_HI):
    return jax.lax.dot_general(a, b, (((1,), (0,)), ((), ())),
                               precision=precision,
                               preferred_element_type=_F32)


def _detect_body(x_ref, xt_ref, w1_ref, w2_ref, w3_ref, w4_ref, b4_ref,
                 out_ref):
    pos = x_ref[...]
    posT = xt_ref[...]
    r0, r1, r2 = pos[0:1, :], pos[1:2, :], pos[2:3, :]
    sq_row = (r0 * r0 + r1 * r1) + r2 * r2
    sq_col = jnp.sum(posT * posT, axis=1, keepdims=True)
    dot = _dot(posT, pos, precision=jax.lax.Precision.DEFAULT)
    d2 = (sq_col + sq_row) - 2.0 * dot

    idx = jax.lax.broadcasted_iota(jnp.int32, (M, M), 1)
    work = d2
    acc = jnp.zeros((M, M), dtype=_F32)
    for _ in range(K_NN):
        mval = jnp.min(work, axis=1, keepdims=True)
        cand = jnp.where(work <= mval, idx, jnp.int32(M))
        jstar = jnp.min(cand, axis=1, keepdims=True)
        sel = idx == jstar
        acc = acc + sel.astype(_F32)
        work = jnp.where(sel, _BIG, work)

    nbr = _dot(acc, posT)
    feats = jnp.concatenate([posT, K_NN * posT - nbr], axis=1)

    lo = jax.lax.Precision.DEFAULT
    h = jnp.maximum(_dot(feats, w1_ref[...], precision=lo), 0.0)
    h = jnp.maximum(_dot(h, w2_ref[...], precision=lo), 0.0)
    h = jnp.maximum(_dot(h, w3_ref[...], precision=lo), 0.0)
    g = jnp.max(h, axis=0, keepdims=True)
    out_ref[...] = _dot(g, w4_ref[...], precision=lo) + b4_ref[...]


def _recip(x):
    r = 1.0 / x
    r = r * (2.0 - x * r)
    r = r * (2.0 - x * r)
    return r


def _rsqrt_acc(x):
    r = jax.lax.rsqrt(x)
    r = r * (1.5 - 0.5 * x * r * r)
    r = r * (1.5 - 0.5 * x * r * r)
    return r


def _bf(x):
    return x.astype(jnp.bfloat16).astype(_F32)


def _jacobi_eig4(nmat):
    a = list(nmat)
    v = [jnp.full_like(a[0], 1.0 if i % 5 == 0 else 0.0) for i in range(16)]
    for _ in range(6):
        for (p, q) in ((0, 1), (0, 2), (0, 3), (1, 2), (1, 3), (2, 3)):
            app, aqq, apq = a[4 * p + p], a[4 * q + q], a[4 * p + q]
            theta = (aqq - app) * _recip(2.0 * apq)
            th2p1 = theta * theta + 1.0
            t_raw = jnp.sign(theta) * _recip(
                jnp.abs(theta) + th2p1 * _rsqrt_acc(th2p1))
            t = jnp.where((jnp.abs(apq) > 0.0) & (th2p1 <= 3.0e38),
                          t_raw, 0.0)
            c = _rsqrt_acc(t * t + 1.0)
            s = t * c
            new_pp = app - t * apq
            new_qq = aqq + t * apq
            for k in range(4):
                if k in (p, q):
                    continue
                akp, akq = a[4 * k + p], a[4 * k + q]
                np_ = c * akp - s * akq
                nq_ = s * akp + c * akq
                a[4 * k + p] = np_
                a[4 * p + k] = np_
                a[4 * k + q] = nq_
                a[4 * q + k] = nq_
            a[4 * p + p] = new_pp
            a[4 * q + q] = new_qq
            zero = jnp.zeros_like(apq)
            a[4 * p + q] = zero
            a[4 * q + p] = zero
            for k in range(4):
                vkp, vkq = v[4 * k + p], v[4 * k + q]
                v[4 * k + p] = c * vkp - s * vkq
                v[4 * k + q] = s * vkp + c * vkq
    diag = [a[0], a[5], a[10], a[15]]
    return diag, v


def _kabsch_body(det_ref, mk_ref, cad_ref, pred_ref, r_ref, t_ref):
    det = det_ref[...]
    src = mk_ref[...]

    scol = []
    sc = []
    for i in range(3):
        row = src[i:i + 1, :]
        m = jnp.mean(row, axis=1, keepdims=True)
        sc.append(m)
        scol.append(row - m)

    tcen = []
    tc = []
    for j in range(3):
        blk = det[:, NKP * j:NKP * (j + 1)]
        m = jnp.mean(blk, axis=1, keepdims=True)
        tc.append(m)
        tcen.append(blk - m)

    sb = [_bf(scol[i]) for i in range(3)]
    tb = [_bf(tcen[j]) for j in range(3)]
    h = [[jnp.sum(sb[i] * tb[j], axis=1, keepdims=True)
          for j in range(3)] for i in range(3)]

    sxx, sxy, sxz = h[0][0], h[0][1], h[0][2]
    syx, syy, syz = h[1][0], h[1][1], h[1][2]
    szx, szy, szz = h[2][0], h[2][1], h[2][2]
    nm = [None] * 16
    nm[0] = sxx + syy + szz
    nm[1] = syz - szy
    nm[2] = szx - sxz
    nm[3] = sxy - syx
    nm[5] = sxx - syy - szz
    nm[6] = sxy + syx
    nm[7] = szx + sxz
    nm[10] = -sxx + syy - szz
    nm[11] = syz + szy
    nm[15] = -sxx - syy + szz
    nm[4], nm[8], nm[12] = nm[1], nm[2], nm[3]
    nm[9], nm[13], nm[14] = nm[6], nm[7], nm[11]

    diag, v = _jacobi_eig4(nm)
    mx = jnp.maximum(jnp.maximum(diag[0], diag[1]),
                     jnp.maximum(diag[2], diag[3]))
    is0 = diag[0] >= mx
    is1 = diag[1] >= mx
    is2 = diag[2] >= mx
    quat = []
    for i in range(4):
        col = jnp.where(is0, v[4 * i + 0],
                        jnp.where(is1, v[4 * i + 1],
                                  jnp.where(is2, v[4 * i + 2], v[4 * i + 3])))
        quat.append(col)
    qw, qx, qy, qz = quat
    nrm = qw * qw + qx * qx + qy * qy + qz * qz
    s2 = 2.0 * _recip(nrm)
    r00 = 1.0 - s2 * (qy * qy + qz * qz)
    r01 = s2 * (qx * qy - qw * qz)
    r02 = s2 * (qx * qz + qw * qy)
    r10 = s2 * (qx * qy + qw * qz)
    r11 = 1.0 - s2 * (qx * qx + qz * qz)
    r12 = s2 * (qy * qz - qw * qx)
    r20 = s2 * (qx * qz - qw * qy)
    r21 = s2 * (qy * qz + qw * qx)
    r22 = 1.0 - s2 * (qx * qx + qy * qy)
    rmat = [[r00, r01, r02], [r10, r11, r12], [r20, r21, r22]]

    rb = [[_bf(rmat[i][j]) for j in range(3)] for i in range(3)]
    scb = [_bf(sc[j]) for j in range(3)]
    tvec = [tc[i] - ((rb[i][0] * scb[0] + rb[i][1] * scb[1])
                     + rb[i][2] * scb[2]) for i in range(3)]

    cad = cad_ref[...]
    cb = [_bf(cad[j:j + 1, :]) for j in range(3)]
    for i in range(3):
        row = ((rb[i][0] * cb[0] + rb[i][1] * cb[1])
               + rb[i][2] * cb[2]) + tvec[i]
        pred_ref[:, i, :] = row
    r_ref[...] = jnp.concatenate(
        [rmat[i][j] for i in range(3) for j in range(3)], axis=1)
    t_ref[...] = jnp.concatenate(tvec, axis=1)


@functools.partial(jax.jit, static_argnames=())
def kernel(input_point_cloud, model_keypoints, cad_models, W1, W2, W3, W4,
           b4):
    x = input_point_cloud.astype(_F32)
    xt = jnp.swapaxes(x, 1, 2)
    b4r = b4.reshape(1, 3 * NKP).astype(_F32)

    wspec = lambda shp: pl.BlockSpec(shp, lambda b: (0, 0))
    detected = pl.pallas_call(
        _detect_body,
        grid=(B,),
        in_specs=[
            pl.BlockSpec((None, 3, M), lambda b: (b, 0, 0)),
            pl.BlockSpec((None, M, 3), lambda b: (b, 0, 0)),
            wspec((6, 32)), wspec((32, 64)), wspec((64, 128)),
            wspec((128, 3 * NKP)), wspec((1, 3 * NKP)),
        ],
        out_specs=pl.BlockSpec((None, 1, 3 * NKP), lambda b: (b, 0, 0)),
        out_shape=jax.ShapeDtypeStruct((B, 1, 3 * NKP), _F32),
        compiler_params=pltpu.CompilerParams(
            dimension_semantics=("parallel",)),
    )(x, xt, W1.astype(_F32), W2.astype(_F32), W3.astype(_F32),
      W4.astype(_F32), b4r)
    detected = detected.reshape(B, 3 * NKP)

    mk = model_keypoints[0].astype(_F32)
    cad = cad_models[0].astype(_F32)
    pred, r9, t3 = pl.pallas_call(
        _kabsch_body,
        in_specs=[pl.BlockSpec((B, 3 * NKP), lambda: (0, 0)),
                  pl.BlockSpec((3, NKP), lambda: (0, 0)),
                  pl.BlockSpec((3, NCAD), lambda: (0, 0))],
        out_specs=[pl.BlockSpec((B, 3, NCAD), lambda: (0, 0, 0)),
                   pl.BlockSpec((B, 9), lambda: (0, 0)),
                   pl.BlockSpec((B, 3), lambda: (0, 0))],
        out_shape=[jax.ShapeDtypeStruct((B, 3, NCAD), _F32),
                   jax.ShapeDtypeStruct((B, 9), _F32),
                   jax.ShapeDtypeStruct((B, 3), _F32)],
    )(detected, mk, cad)

    return (pred, detected.reshape(B, 3, NKP), r9.reshape(B, 3, 3),
            t3.reshape(B, 3, 1))
